# baseline (speedup 1.0000x reference)
import functools

import jax
import jax.numpy as jnp
from jax import lax
from jax.experimental import pallas as pl
from jax.experimental.pallas import tpu as pltpu
from jax.experimental.pallas import tpu_sc as plsc

B = 16384
C = 3
NUM_SUBCORES = 16
LANES = 16
NW = NUM_SUBCORES
BPW = B // NW
STEPS = BPW // LANES

_MESH = plsc.VectorSubcoreMesh(
    core_axis_name="c", subcore_axis_name="s",
    num_cores=1, num_subcores=NUM_SUBCORES,
)


@functools.partial(
    pl.kernel,
    out_type=[jax.ShapeDtypeStruct((NW, LANES), jnp.float32),
              jax.ShapeDtypeStruct((1,), jnp.float32)],
    mesh=_MESH,
    scratch_types=[
        pltpu.VMEM((C, BPW), jnp.float32),
        pltpu.VMEM((BPW,), jnp.int32),
        pltpu.VMEM((BPW,), jnp.int32),
        pltpu.VMEM((LANES,), jnp.float32),
        pltpu.VMEM((LANES,), jnp.float32),
        pltpu.VMEM((NW, LANES), jnp.float32),
        pltpu.SemaphoreType.DMA,
        pltpu.SemaphoreType.DMA,
        pltpu.SemaphoreType.DMA,
        pltpu.SemaphoreType.DMA,
        pltpu.SemaphoreType.DMA,
    ],
    compiler_params=pltpu.CompilerParams(needs_layout_passes=False),
)
def _dro_loss_sc(logits_hbm, labels_hbm, groups_hbm, wts_hbm,
                 parts_hbm, out_hbm,
                 logits_v, labels_v, groups_v, wts_v, acc_v, red_v,
                 wsem, sem0, sem1, sem2, sem3):
    sid = lax.axis_index("s")
    base = sid * BPW
    wts_v[...] = jnp.zeros((LANES,), jnp.float32)
    wcopy = pltpu.async_copy(wts_hbm, wts_v.at[pl.ds(0, 2)], wsem)
    NCHUNK = 4
    CW = BPW // NCHUNK
    sems = [sem0, sem1, sem2, sem3]
    copies = []
    for c in range(NCHUNK):
        col = c * CW
        copies.append([
            pltpu.async_copy(logits_hbm.at[:, pl.ds(base + col, CW)],
                             logits_v.at[:, pl.ds(col, CW)], sems[c]),
            pltpu.async_copy(labels_hbm.at[pl.ds(base + col, CW)],
                             labels_v.at[pl.ds(col, CW)], sems[c]),
            pltpu.async_copy(groups_hbm.at[pl.ds(base + col, CW)],
                             groups_v.at[pl.ds(col, CW)], sems[c]),
        ])
    wcopy.wait()

    acc = jnp.zeros((LANES,), jnp.float32)
    for j in range(STEPS):
        off = j * LANES
        if off % CW == 0:
            for cp in copies[off // CW]:
                cp.wait()
        l0 = logits_v[0, pl.ds(off, LANES)]
        l1 = logits_v[1, pl.ds(off, LANES)]
        l2 = logits_v[2, pl.ds(off, LANES)]
        lab = labels_v[pl.ds(off, LANES)]
        gid = groups_v[pl.ds(off, LANES)]
        m = jnp.maximum(l0, jnp.maximum(l1, l2))
        s = jnp.exp(l0 - m) + jnp.exp(l1 - m) + jnp.exp(l2 - m)
        z = (s - 1.0) / (s + 1.0)
        z2 = z * z
        p = jnp.float32(1.0 / 9.0)
        for coef in (1.0 / 7.0, 1.0 / 5.0, 1.0 / 3.0, 1.0):
            p = p * z2 + jnp.float32(coef)
        log_s = (2.0 * z) * p
        l_lab = jnp.where(lab == 0, l0, jnp.where(lab == 1, l1, l2))
        w = plsc.load_gather(wts_v, [gid])
        acc = acc + (log_s + (m - l_lab)) * w

    acc_v[...] = acc
    pltpu.sync_copy(acc_v, parts_hbm.at[sid])
    plsc.subcore_barrier()

    @pl.when(sid == 0)
    def _():
        pltpu.sync_copy(parts_hbm, red_v)
        tot = jnp.zeros((LANES,), jnp.float32)
        for t in range(NW):
            tot = tot + red_v[t, :]
        total = jnp.sum(tot)
        acc_v[...] = jnp.full((LANES,), total, jnp.float32)
        pltpu.sync_copy(acc_v.at[pl.ds(0, 1)], out_hbm)


def kernel(logits, labels, group_indices, subgroup_indices, group_weights):
    del subgroup_indices
    _, total = _dro_loss_sc(
        logits.astype(jnp.float32).T,
        labels.astype(jnp.int32),
        group_indices.astype(jnp.int32),
        group_weights.astype(jnp.float32),
    )
    return total.reshape(())


# speedup vs baseline: 1.1450x; 1.1450x over previous
import functools

import jax
import jax.numpy as jnp
from jax import lax
from jax.experimental import pallas as pl
from jax.experimental.pallas import tpu as pltpu
from jax.experimental.pallas import tpu_sc as plsc

B = 16384
C = 3
NUM_SUBCORES = 16
LANES = 16
NW = NUM_SUBCORES
BPW = B // NW
STEPS = BPW // LANES

_MESH = plsc.VectorSubcoreMesh(
    core_axis_name="c", subcore_axis_name="s",
    num_cores=1, num_subcores=NUM_SUBCORES,
)


@functools.partial(
    pl.kernel,
    out_type=[jax.ShapeDtypeStruct((NW, LANES), jnp.float32),
              jax.ShapeDtypeStruct((1,), jnp.float32)],
    mesh=_MESH,
    scratch_types=[
        pltpu.VMEM((C, BPW), jnp.float32),
        pltpu.VMEM((BPW,), jnp.int32),
        pltpu.VMEM((BPW,), jnp.int32),
        pltpu.VMEM((LANES,), jnp.float32),
        pltpu.VMEM((LANES,), jnp.float32),
        pltpu.VMEM((NW, LANES), jnp.float32),
        pltpu.SemaphoreType.DMA,
        pltpu.SemaphoreType.DMA,
        pltpu.SemaphoreType.DMA,
        pltpu.SemaphoreType.DMA,
        pltpu.SemaphoreType.DMA,
    ],
    compiler_params=pltpu.CompilerParams(needs_layout_passes=False),
)
def _dro_loss_sc(logits_hbm, labels_hbm, groups_hbm, wts_hbm,
                 parts_hbm, out_hbm,
                 logits_v, labels_v, groups_v, wts_v, acc_v, red_v,
                 wsem, sem0, sem1, sem2, sem3):
    sid = lax.axis_index("s")
    base = sid * BPW
    wts_v[...] = jnp.zeros((LANES,), jnp.float32)
    wcopy = pltpu.async_copy(wts_hbm, wts_v.at[pl.ds(0, 2)], wsem)
    copies = [
        pltpu.async_copy(logits_hbm.at[:, pl.ds(base, BPW)], logits_v, sem0),
        pltpu.async_copy(labels_hbm.at[pl.ds(base, BPW)], labels_v, sem1),
        pltpu.async_copy(groups_hbm.at[pl.ds(base, BPW)], groups_v, sem2),
    ]
    wcopy.wait()
    for cp in copies:
        cp.wait()

    UNROLL = 4

    def _step(j, acc):
        acc_u = acc
        for u in range(UNROLL):
            off = j * (UNROLL * LANES) + u * LANES
            l0 = logits_v[0, pl.ds(off, LANES)]
            l1 = logits_v[1, pl.ds(off, LANES)]
            l2 = logits_v[2, pl.ds(off, LANES)]
            lab = labels_v[pl.ds(off, LANES)]
            gid = groups_v[pl.ds(off, LANES)]
            m = jnp.maximum(l0, jnp.maximum(l1, l2))
            s = jnp.exp(l0 - m) + jnp.exp(l1 - m) + jnp.exp(l2 - m)
            z = (s - 1.0) / (s + 1.0)
            z2 = z * z
            p = jnp.float32(1.0 / 9.0)
            for coef in (1.0 / 7.0, 1.0 / 5.0, 1.0 / 3.0, 1.0):
                p = p * z2 + jnp.float32(coef)
            log_s = (2.0 * z) * p
            l_lab = jnp.where(lab == 0, l0, jnp.where(lab == 1, l1, l2))
            w = plsc.load_gather(wts_v, [gid])
            acc_u = acc_u + (log_s + (m - l_lab)) * w
        return acc_u

    acc = lax.fori_loop(0, STEPS // UNROLL, _step,
                        jnp.zeros((LANES,), jnp.float32))

    acc_v[...] = acc
    pltpu.sync_copy(acc_v, parts_hbm.at[sid])
    plsc.subcore_barrier()

    @pl.when(sid == 0)
    def _():
        pltpu.sync_copy(parts_hbm, red_v)
        tot = jnp.zeros((LANES,), jnp.float32)
        for t in range(NW):
            tot = tot + red_v[t, :]
        total = jnp.sum(tot)
        acc_v[...] = jnp.full((LANES,), total, jnp.float32)
        pltpu.sync_copy(acc_v.at[pl.ds(0, 1)], out_hbm)


def kernel(logits, labels, group_indices, subgroup_indices, group_weights):
    del subgroup_indices
    _, total = _dro_loss_sc(
        logits.astype(jnp.float32).T,
        labels.astype(jnp.int32),
        group_indices.astype(jnp.int32),
        group_weights.astype(jnp.float32),
    )
    return total.reshape(())
